# baseline (speedup 1.0000x reference)
import functools

import jax
import jax.numpy as jnp
from jax import lax
from jax.experimental import pallas as pl
from jax.experimental.pallas import tpu as pltpu
from jax.experimental.pallas import tpu_sc as plsc

N = 10000
E = 320000
NB, NC, ND = 10, 1000, 128
H1, O1 = 8, 8
H2, O2 = 1, 128
F1 = H1 * O1
F2 = H2 * O2
SLOPE = 0.2

NCORES, NSUB = 2, 16
NTILES = NCORES * NSUB
C1 = 80
NCHUNKS1 = E // C1
CPT1 = NCHUNKS1 // NTILES
C2 = 64
NCHUNKS2 = E // C2
CPT2 = -(-NCHUNKS2 // NTILES)
ZROWS = 80
NZC = N // ZROWS
DPAD = ZROWS * 128


def _leaky_exp(a):
    return jnp.exp(jnp.maximum(a, SLOPE * a))


def _tc1_body(x_ref, seq_ref, W1_ref, As_ref, Ad_ref, table_ref, adst_ref):
    xv = x_ref[...] * seq_ref[...]
    h = jnp.dot(xv, W1_ref[...], preferred_element_type=jnp.float32)
    asrc = jnp.dot(h, As_ref[...], preferred_element_type=jnp.float32)
    adst = jnp.dot(h, Ad_ref[...], preferred_element_type=jnp.float32)
    pad = jnp.zeros((h.shape[0], 128 - F1 - 2 * H1), jnp.float32)
    table_ref[...] = jnp.concatenate([h, asrc, adst, pad], axis=1)
    adst_ref[...] = adst


def _tc2_body(acc_ref, table_ref, W2_ref, a2s_ref, a2d_ref,
              b1_ref, Eexp_ref, table2_ref, as2_ref, ad2_ref):
    t = table_ref[...]
    h = t[:, :F1]
    asrc = t[:, F1:F1 + H1]
    adst = t[:, F1 + H1:F1 + 2 * H1]
    exs = _leaky_exp(asrc + adst)
    s = acc_ref[0] + acc_ref[1]
    Eexp = Eexp_ref[...]
    num = s[:, :F1] + h * jnp.dot(exs, Eexp, preferred_element_type=jnp.float32)
    den = s[:, F1:F1 + H1] + exs
    deninv = 1.0 / (jnp.dot(den, Eexp, preferred_element_type=jnp.float32) + 1e-16)
    o1 = num * deninv + b1_ref[...]
    e = jnp.where(o1 > 0, o1, jnp.exp(jnp.minimum(o1, 0.0)) - 1.0)
    h2 = jnp.dot(e, W2_ref[...], preferred_element_type=jnp.float32)
    table2_ref[...] = h2
    as2_ref[...] = jnp.dot(h2, a2s_ref[...], preferred_element_type=jnp.float32)
    ad2_ref[...] = jnp.dot(h2, a2d_ref[...], preferred_element_type=jnp.float32)


def _tcden_body(denp_ref, out_ref):
    out_ref[...] = jnp.sum(denp_ref[...], axis=0)


def _tc3_body(acc_ref, table2_ref, as2_ref, ad2_ref, den_ref, b2_ref, out_ref):
    h2 = table2_ref[...]
    exs = _leaky_exp(as2_ref[...] + ad2_ref[...])
    s = acc_ref[0] + acc_ref[1]
    num = s + h2 * exs
    den = den_ref[...] + exs
    out_ref[...] = num / (den + 1e-16) + b2_ref[...]


def _tc_call(body, in_specs, out_specs, out_shapes, args, grid):
    return pl.pallas_call(
        body,
        grid=grid,
        in_specs=in_specs,
        out_specs=out_specs,
        out_shape=out_shapes,
    )(*args)


def _rows_spec(width, blk=NC):
    return pl.BlockSpec((blk, width), lambda i: (i, 0))


def _full_spec(shape):
    return pl.BlockSpec(shape, lambda i: tuple(0 for _ in shape))


def _acc_spec(width, blk=NC):
    return pl.BlockSpec((2, blk, width), lambda i: (0, i, 0))


def _zero_rows(buf, nrows):
    def zb(i, c):
        buf[i // 8, pl.ds((i % 8) * 16, 16)] = jnp.zeros((16,), jnp.float32)
        return c
    lax.fori_loop(0, nrows * 8, zb, 0)


def _zero_acc(sid, zsrc, acc):
    def zacc(k, c):
        m = sid + k * NSUB
        @pl.when(m < NZC)
        def _():
            pltpu.sync_copy(zsrc, acc.at[pl.ds(m * ZROWS, ZROWS)])
        return c
    lax.fori_loop(0, -(-NZC // NSUB), zacc, 0)


def _drain_acc(cid, sid, acc, out_hbm):
    def drain(k, c):
        m = sid + k * NSUB
        @pl.when(m < NZC)
        def _():
            pltpu.sync_copy(acc.at[pl.ds(m * ZROWS, ZROWS)],
                            out_hbm.at[cid, pl.ds(m * ZROWS, ZROWS)])
        return c
    lax.fori_loop(0, -(-NZC // NSUB), drain, 0)


def _make_sc1():
    def body(srcf, dstf, table_hbm, adst_hbm, out_hbm,
             sidx0, sidx1, didx0, didx1, d16_0, d16_1,
             rows0, rows1, drows0, drows1, adsp, acc,
             isem0, isem1, rsem0, rsem1, dsem0, dsem1):
        cid = lax.axis_index("c")
        sid = lax.axis_index("s")
        wid = cid * NSUB + sid
        lanes = lax.iota(jnp.int32, 16)
        hi = lanes >> 3
        lo = lanes & 7
        sidx = (sidx0, sidx1)
        didx = (didx0, didx1)
        d16 = (d16_0, d16_1)
        rows = (rows0, rows1)
        drows = (drows0, drows1)
        isem = (isem0, isem1)
        rsem = (rsem0, rsem1)
        dsem = (dsem0, dsem1)

        def issue_idx(j, b):
            c = (wid + j * NTILES) * C1
            pltpu.async_copy(srcf.at[pl.ds(c, C1)], sidx[b], isem[b])
            pltpu.async_copy(dstf.at[pl.ds(c, C1)], didx[b], isem[b])

        def wait_idx(j, b):
            c = (wid + j * NTILES) * C1
            pltpu.make_async_copy(srcf.at[pl.ds(c, C1)], sidx[b], isem[b]).wait()
            pltpu.make_async_copy(dstf.at[pl.ds(c, C1)], didx[b], isem[b]).wait()

        def prep_d16(b):
            def p(g, c):
                d16[b][pl.ds(16 * g, 16)] = didx[b][pl.ds(16 * g, 16)] >> 4
                return c
            lax.fori_loop(0, C1 // 16, p, 0)

        def issue_rows(b):
            pltpu.async_copy(table_hbm.at[sidx[b]], rows[b], rsem[b])
            pltpu.async_copy(adsp.at[d16[b]], drows[b], dsem[b])

        def wait_rows(b):
            pltpu.make_async_copy(table_hbm.at[sidx[b]], rows[b], rsem[b]).wait()
            pltpu.make_async_copy(adsp.at[d16[b]], drows[b], dsem[b]).wait()

        _zero_rows(rows0, ZROWS)
        _zero_acc(sid, rows0, acc)
        @pl.when(sid == 0)
        def _():
            pltpu.sync_copy(adst_hbm, adsp)
        issue_idx(0, 0)
        wait_idx(0, 0)
        issue_idx(1, 1)
        plsc.subcore_barrier()
        prep_d16(0)
        issue_rows(0)

        def step(j, b):
            @pl.when(j < CPT1)
            def _():
                wait_rows(b)
            @pl.when(j + 1 < CPT1)
            def _():
                wait_idx(j + 1, 1 - b)
                prep_d16(1 - b)
                issue_rows(1 - b)
            @pl.when(j < CPT1)
            def _():
                rb, db = rows[b], drows[b]

                def attn(p, c):
                    c0 = 2 * p
                    dv = plsc.load_gather(didx[b], [c0 + hi])
                    av = plsc.load_gather(rb, [c0 + hi, F1 + lo])
                    bv = plsc.load_gather(db, [c0 + hi, ((dv & 15) << 3) + lo])
                    ex = _leaky_exp(av + bv)
                    plsc.store_scatter(rb, [c0 + hi, F1 + lo], ex)
                    return c
                lax.fori_loop(0, C1 // 2, attn, 0)

                def msgs(c, c2):
                    cc = jnp.full((16,), c, jnp.int32)
                    for t in range(F1 // 16):
                        exb = plsc.load_gather(rb, [cc, F1 + 2 * t + hi])
                        rb[c, pl.ds(16 * t, 16)] = (
                            rb[c, pl.ds(16 * t, 16)] * exb)
                    return c2
                lax.fori_loop(0, C1, msgs, 0)

                pltpu.sync_copy(rb, acc.at[didx[b]], add=True)
            @pl.when(j + 2 < CPT1)
            def _():
                issue_idx(j + 2, b)

        def kbody(k, carry):
            step(2 * k, 0)
            step(2 * k + 1, 1)
            return carry
        lax.fori_loop(0, (CPT1 + 1) // 2, kbody, 0)

        plsc.subcore_barrier()
        _drain_acc(cid, sid, acc, out_hbm)

    mesh = plsc.VectorSubcoreMesh(core_axis_name="c", subcore_axis_name="s")
    return functools.partial(
        pl.kernel,
        out_type=jax.ShapeDtypeStruct((2, N, 128), jnp.float32),
        mesh=mesh,
        compiler_params=pltpu.CompilerParams(
            needs_layout_passes=False, use_tc_tiling_on_sc=False),
        scratch_types=[
            pltpu.VMEM((C1,), jnp.int32),
            pltpu.VMEM((C1,), jnp.int32),
            pltpu.VMEM((C1,), jnp.int32),
            pltpu.VMEM((C1,), jnp.int32),
            pltpu.VMEM((C1,), jnp.int32),
            pltpu.VMEM((C1,), jnp.int32),
            pltpu.VMEM((C1, 128), jnp.float32),
            pltpu.VMEM((C1, 128), jnp.float32),
            pltpu.VMEM((C1, 128), jnp.float32),
            pltpu.VMEM((C1, 128), jnp.float32),
            pltpu.VMEM_SHARED((N * H1 // 128, 128), jnp.float32),
            pltpu.VMEM_SHARED((N, 128), jnp.float32),
            pltpu.SemaphoreType.DMA,
            pltpu.SemaphoreType.DMA,
            pltpu.SemaphoreType.DMA,
            pltpu.SemaphoreType.DMA,
            pltpu.SemaphoreType.DMA,
            pltpu.SemaphoreType.DMA,
        ],
    )(body)


def _make_sc2():
    def body(srcf, dstf, table_hbm, as2_hbm, ad2_hbm,
             out_hbm, den_hbm,
             sidx0, sidx1, didx0, didx1, rows0, rows1, asv, adv, den,
             exbuf, acc, isem0, isem1, rsem0, rsem1):
        cid = lax.axis_index("c")
        sid = lax.axis_index("s")
        wid = cid * NSUB + sid
        lanes = lax.iota(jnp.int32, 16)
        sidx = (sidx0, sidx1)
        didx = (didx0, didx1)
        rows = (rows0, rows1)
        isem = (isem0, isem1)
        rsem = (rsem0, rsem1)

        def issue_idx(j, b):
            c = (wid + j * NTILES) * C2
            pltpu.async_copy(srcf.at[pl.ds(c, C2)], sidx[b], isem[b])
            pltpu.async_copy(dstf.at[pl.ds(c, C2)], didx[b], isem[b])

        def wait_idx(j, b):
            c = (wid + j * NTILES) * C2
            pltpu.make_async_copy(srcf.at[pl.ds(c, C2)], sidx[b], isem[b]).wait()
            pltpu.make_async_copy(dstf.at[pl.ds(c, C2)], didx[b], isem[b]).wait()

        def issue_rows(b):
            pltpu.async_copy(table_hbm.at[sidx[b]], rows[b], rsem[b])

        def wait_rows(b):
            pltpu.make_async_copy(table_hbm.at[sidx[b]], rows[b], rsem[b]).wait()

        def valid(j):
            return (wid + j * NTILES) < NCHUNKS2

        _zero_rows(den, ZROWS)
        pltpu.sync_copy(as2_hbm, asv)
        pltpu.sync_copy(ad2_hbm, adv)
        _zero_acc(sid, den, acc)
        @pl.when(valid(0))
        def _():
            issue_idx(0, 0)
            wait_idx(0, 0)
            issue_rows(0)
        @pl.when(valid(1))
        def _():
            issue_idx(1, 1)
        plsc.subcore_barrier()

        def step(j, b):
            @pl.when(valid(j))
            def _():
                wait_rows(b)
            @pl.when(valid(j + 1))
            def _():
                wait_idx(j + 1, 1 - b)
                issue_rows(1 - b)
            @pl.when(valid(j))
            def _():
                rb = rows[b]

                def attn(g, c):
                    c0 = 16 * g
                    sv = plsc.load_gather(sidx[b], [c0 + lanes])
                    dv = plsc.load_gather(didx[b], [c0 + lanes])
                    av = plsc.load_gather(asv, [sv >> 7, sv & 127])
                    bv = plsc.load_gather(adv, [dv >> 7, dv & 127])
                    ex = _leaky_exp(av + bv)
                    plsc.addupdate_scatter(den, [dv >> 7, dv & 127], ex)
                    exbuf[pl.ds(c0, 16)] = ex
                    return c
                lax.fori_loop(0, C2 // 16, attn, 0)

                def msgs(c, c2):
                    exb = plsc.load_gather(exbuf, [jnp.full((16,), c, jnp.int32)])
                    for t in range(F2 // 16):
                        rb[c, pl.ds(16 * t, 16)] = (
                            rb[c, pl.ds(16 * t, 16)] * exb)
                    return c2
                lax.fori_loop(0, C2, msgs, 0)

                pltpu.sync_copy(rb, acc.at[didx[b]], add=True)
            @pl.when(valid(j + 2))
            def _():
                issue_idx(j + 2, b)

        def kbody(k, carry):
            step(2 * k, 0)
            step(2 * k + 1, 1)
            return carry
        lax.fori_loop(0, (CPT2 + 1) // 2, kbody, 0)

        plsc.subcore_barrier()
        _drain_acc(cid, sid, acc, out_hbm)
        pltpu.sync_copy(den, den_hbm.at[wid])

    mesh = plsc.VectorSubcoreMesh(core_axis_name="c", subcore_axis_name="s")
    return functools.partial(
        pl.kernel,
        out_type=[jax.ShapeDtypeStruct((2, N, 128), jnp.float32),
                  jax.ShapeDtypeStruct((NTILES, ZROWS, 128), jnp.float32)],
        mesh=mesh,
        compiler_params=pltpu.CompilerParams(
            needs_layout_passes=False, use_tc_tiling_on_sc=False),
        scratch_types=[
            pltpu.VMEM((C2,), jnp.int32),
            pltpu.VMEM((C2,), jnp.int32),
            pltpu.VMEM((C2,), jnp.int32),
            pltpu.VMEM((C2,), jnp.int32),
            pltpu.VMEM((C2, 128), jnp.float32),
            pltpu.VMEM((C2, 128), jnp.float32),
            pltpu.VMEM((ZROWS, 128), jnp.float32),
            pltpu.VMEM((ZROWS, 128), jnp.float32),
            pltpu.VMEM((ZROWS, 128), jnp.float32),
            pltpu.VMEM((C2,), jnp.float32),
            pltpu.VMEM_SHARED((N, 128), jnp.float32),
            pltpu.SemaphoreType.DMA,
            pltpu.SemaphoreType.DMA,
            pltpu.SemaphoreType.DMA,
            pltpu.SemaphoreType.DMA,
        ],
    )(body)


@functools.lru_cache(maxsize=None)
def _sc_kernels():
    return (_make_sc1(), _make_sc2())


def kernel(x, seq, edges, W1, att_src1, att_dst1, bias1,
           W2, att_src2, att_dst2, bias2):
    _sc_layer1, _sc_layer2 = _sc_kernels()
    xf = x.reshape(N, ND)
    edges = edges.astype(jnp.int32)
    srcf = edges[0].reshape(E)
    dstf = edges[1].reshape(E)

    k1 = jnp.arange(F1)
    j1 = jnp.arange(H1)
    As1 = jnp.where((k1[:, None] // O1) == j1[None, :],
                    att_src1.reshape(F1)[:, None], 0.0)
    Ad1 = jnp.where((k1[:, None] // O1) == j1[None, :],
                    att_dst1.reshape(F1)[:, None], 0.0)
    a2s = att_src2.reshape(F2, 1)
    a2d = att_dst2.reshape(F2, 1)
    Eexp = ((jnp.arange(F1)[None, :] // O1) == jnp.arange(H1)[:, None]
            ).astype(jnp.float32)
    b1 = bias1.reshape(1, F1)
    b2 = bias2.reshape(1, F2)

    table1, adst1 = _tc_call(
        _tc1_body,
        [_rows_spec(ND), _rows_spec(1), _full_spec((ND, F1)),
         _full_spec((F1, H1)), _full_spec((F1, H1))],
        [_rows_spec(128), _rows_spec(H1)],
        [jax.ShapeDtypeStruct((N, 128), jnp.float32),
         jax.ShapeDtypeStruct((N, H1), jnp.float32)],
        (xf, seq, W1, As1, Ad1),
        grid=(N // NC,),
    )

    acc1 = _sc_layer1(srcf, dstf, table1,
                      adst1.reshape(N * H1 // 128, 128))

    table2, as2, ad2 = _tc_call(
        _tc2_body,
        [_acc_spec(128), _rows_spec(128), _full_spec((F1, F2)),
         _full_spec((F2, 1)), _full_spec((F2, 1)), _full_spec((1, F1)),
         _full_spec((H1, F1))],
        [_rows_spec(128), _rows_spec(1), _rows_spec(1)],
        [jax.ShapeDtypeStruct((N, 128), jnp.float32),
         jax.ShapeDtypeStruct((N, 1), jnp.float32),
         jax.ShapeDtypeStruct((N, 1), jnp.float32)],
        (acc1, table1, W2, a2s, a2d, b1, Eexp),
        grid=(N // NC,),
    )

    as2p = jnp.pad(as2.reshape(N), (0, DPAD - N)).reshape(ZROWS, 128)
    ad2p = jnp.pad(ad2.reshape(N), (0, DPAD - N)).reshape(ZROWS, 128)
    acc2, denp = _sc_layer2(srcf, dstf, table2, as2p, ad2p)

    (denr,) = _tc_call(
        _tcden_body,
        [_full_spec((NTILES, ZROWS, 128))],
        [_full_spec((ZROWS, 128))],
        [jax.ShapeDtypeStruct((ZROWS, 128), jnp.float32)],
        (denp,),
        grid=(1,),
    )
    den_n = denr.reshape(DPAD)[:N].reshape(N, 1)

    (out,) = _tc_call(
        _tc3_body,
        [_acc_spec(128), _rows_spec(128), _rows_spec(1), _rows_spec(1),
         _rows_spec(1), _full_spec((1, F2))],
        [_rows_spec(F2)],
        [jax.ShapeDtypeStruct((N, F2), jnp.float32)],
        (acc2, table2, as2, ad2, den_n, b2),
        grid=(N // NC,),
    )
    return out.reshape(NB, NC, ND)


# speedup vs baseline: 1.1024x; 1.1024x over previous
import functools

import jax
import jax.numpy as jnp
from jax import lax
from jax.experimental import pallas as pl
from jax.experimental.pallas import tpu as pltpu
from jax.experimental.pallas import tpu_sc as plsc

N = 10000
E = 320000
NB, NC, ND = 10, 1000, 128
H1, O1 = 8, 8
H2, O2 = 1, 128
F1 = H1 * O1
F2 = H2 * O2
SLOPE = 0.2

NCORES, NSUB = 2, 16
NTILES = NCORES * NSUB
C1 = 80
NCHUNKS1 = E // C1
CPT1 = NCHUNKS1 // NTILES
C2 = 64
NCHUNKS2 = E // C2
CPT2 = -(-NCHUNKS2 // NTILES)
ZROWS = 80
NZC = N // ZROWS
DPAD = ZROWS * 128


def _leaky_exp(a):
    return jnp.exp(jnp.maximum(a, SLOPE * a))


def _tc1_body(x_ref, seq_ref, W1_ref, As_ref, Ad_ref, table_ref):
    xv = x_ref[...] * seq_ref[...]
    h = jnp.dot(xv, W1_ref[...], preferred_element_type=jnp.float32)
    asrc = jnp.dot(h, As_ref[...], preferred_element_type=jnp.float32)
    adst = jnp.dot(h, Ad_ref[...], preferred_element_type=jnp.float32)
    pad = jnp.zeros((h.shape[0], 128 - F1 - 2 * H1), jnp.float32)
    table_ref[...] = jnp.concatenate([h, asrc, adst, pad], axis=1)


def _tc2_body(acc_ref, table_ref, W2_ref, a2s_ref, a2d_ref,
              b1_ref, Eexp_ref, table2_ref, as2_ref, ad2_ref):
    t = table_ref[...]
    h = t[:, :F1]
    asrc = t[:, F1:F1 + H1]
    adst = t[:, F1 + H1:F1 + 2 * H1]
    exs = _leaky_exp(asrc + adst)
    s = acc_ref[0] + acc_ref[1]
    Eexp = Eexp_ref[...]
    num = s[:, :F1] + h * jnp.dot(exs, Eexp, preferred_element_type=jnp.float32)
    den = s[:, F1:F1 + H1] + exs
    deninv = 1.0 / (jnp.dot(den, Eexp, preferred_element_type=jnp.float32) + 1e-16)
    o1 = num * deninv + b1_ref[...]
    e = jnp.where(o1 > 0, o1, jnp.exp(jnp.minimum(o1, 0.0)) - 1.0)
    h2 = jnp.dot(e, W2_ref[...], preferred_element_type=jnp.float32)
    table2_ref[...] = h2
    as2_ref[...] = jnp.dot(h2, a2s_ref[...], preferred_element_type=jnp.float32)
    ad2_ref[...] = jnp.dot(h2, a2d_ref[...], preferred_element_type=jnp.float32)


def _tcden_body(denp_ref, out_ref):
    out_ref[...] = jnp.sum(denp_ref[...], axis=0)


def _tc3_body(acc_ref, table2_ref, as2_ref, ad2_ref, den_ref, b2_ref, out_ref):
    h2 = table2_ref[...]
    exs = _leaky_exp(as2_ref[...] + ad2_ref[...])
    s = acc_ref[0] + acc_ref[1]
    num = s + h2 * exs
    den = den_ref[...] + exs
    out_ref[...] = num / (den + 1e-16) + b2_ref[...]


def _tc_call(body, in_specs, out_specs, out_shapes, args, grid):
    return pl.pallas_call(
        body,
        grid=grid,
        in_specs=in_specs,
        out_specs=out_specs,
        out_shape=out_shapes,
    )(*args)


def _rows_spec(width, blk=NC):
    return pl.BlockSpec((blk, width), lambda i: (i, 0))


def _full_spec(shape):
    return pl.BlockSpec(shape, lambda i: tuple(0 for _ in shape))


def _acc_spec(width, blk=NC):
    return pl.BlockSpec((2, blk, width), lambda i: (0, i, 0))


def _zero_rows(buf, nrows):
    def zb(i, c):
        buf[i // 8, pl.ds((i % 8) * 16, 16)] = jnp.zeros((16,), jnp.float32)
        return c
    lax.fori_loop(0, nrows * 8, zb, 0)


def _zero_acc(sid, zsrc, acc):
    def zacc(k, c):
        m = sid + k * NSUB
        @pl.when(m < NZC)
        def _():
            pltpu.sync_copy(zsrc, acc.at[pl.ds(m * ZROWS, ZROWS)])
        return c
    lax.fori_loop(0, -(-NZC // NSUB), zacc, 0)


def _drain_acc(cid, sid, acc, out_hbm):
    def drain(k, c):
        m = sid + k * NSUB
        @pl.when(m < NZC)
        def _():
            pltpu.sync_copy(acc.at[pl.ds(m * ZROWS, ZROWS)],
                            out_hbm.at[cid, pl.ds(m * ZROWS, ZROWS)])
        return c
    lax.fori_loop(0, -(-NZC // NSUB), drain, 0)


def _make_sc1():
    def body(srcf, dstf, table_hbm, out_hbm,
             sidx0, sidx1, didx0, didx1,
             rows0, rows1, drows0, drows1, acc,
             isem0, isem1, rsem0, rsem1, dsem0, dsem1):
        cid = lax.axis_index("c")
        sid = lax.axis_index("s")
        wid = cid * NSUB + sid
        lanes = lax.iota(jnp.int32, 16)
        hi = lanes >> 3
        lo = lanes & 7
        sidx = (sidx0, sidx1)
        didx = (didx0, didx1)
        rows = (rows0, rows1)
        drows = (drows0, drows1)
        isem = (isem0, isem1)
        rsem = (rsem0, rsem1)
        dsem = (dsem0, dsem1)

        def issue_idx(j, b):
            c = (wid + j * NTILES) * C1
            pltpu.async_copy(srcf.at[pl.ds(c, C1)], sidx[b], isem[b])
            pltpu.async_copy(dstf.at[pl.ds(c, C1)], didx[b], isem[b])

        def wait_idx(j, b):
            c = (wid + j * NTILES) * C1
            pltpu.make_async_copy(srcf.at[pl.ds(c, C1)], sidx[b], isem[b]).wait()
            pltpu.make_async_copy(dstf.at[pl.ds(c, C1)], didx[b], isem[b]).wait()

        def issue_rows(b):
            pltpu.async_copy(table_hbm.at[sidx[b]], rows[b], rsem[b])
            pltpu.async_copy(table_hbm.at[didx[b]], drows[b], dsem[b])

        def wait_rows(b):
            pltpu.make_async_copy(table_hbm.at[sidx[b]], rows[b], rsem[b]).wait()
            pltpu.make_async_copy(table_hbm.at[didx[b]], drows[b], dsem[b]).wait()

        _zero_rows(rows0, ZROWS)
        _zero_acc(sid, rows0, acc)
        issue_idx(0, 0)
        wait_idx(0, 0)
        issue_rows(0)
        issue_idx(1, 1)
        plsc.subcore_barrier()

        def step(j, b):
            @pl.when(j < CPT1)
            def _():
                wait_rows(b)
            @pl.when(j + 1 < CPT1)
            def _():
                wait_idx(j + 1, 1 - b)
                issue_rows(1 - b)
            @pl.when(j < CPT1)
            def _():
                rb, db = rows[b], drows[b]

                def attn(p, c):
                    for q in range(2):
                        c0 = 4 * p + 2 * q
                        av = plsc.load_gather(rb, [c0 + hi, F1 + lo])
                        bv = plsc.load_gather(db, [c0 + hi, F1 + H1 + lo])
                        ex = _leaky_exp(av + bv)
                        plsc.store_scatter(rb, [c0 + hi, F1 + lo], ex)
                    return c
                lax.fori_loop(0, C1 // 4, attn, 0)

                def msgs(m, c2):
                    for q in range(2):
                        c = 2 * m + q
                        cc = jnp.full((16,), c, jnp.int32)
                        for t in range(F1 // 16):
                            exb = plsc.load_gather(rb, [cc, F1 + 2 * t + hi])
                            rb[c, pl.ds(16 * t, 16)] = (
                                rb[c, pl.ds(16 * t, 16)] * exb)
                    return c2
                lax.fori_loop(0, C1 // 2, msgs, 0)

                pltpu.sync_copy(rb, acc.at[didx[b]], add=True)
            @pl.when(j + 2 < CPT1)
            def _():
                issue_idx(j + 2, b)

        def kbody(k, carry):
            step(2 * k, 0)
            step(2 * k + 1, 1)
            return carry
        lax.fori_loop(0, (CPT1 + 1) // 2, kbody, 0)

        plsc.subcore_barrier()
        _drain_acc(cid, sid, acc, out_hbm)

    mesh = plsc.VectorSubcoreMesh(core_axis_name="c", subcore_axis_name="s")
    return functools.partial(
        pl.kernel,
        out_type=jax.ShapeDtypeStruct((2, N, 128), jnp.float32),
        mesh=mesh,
        compiler_params=pltpu.CompilerParams(
            needs_layout_passes=False, use_tc_tiling_on_sc=False),
        scratch_types=[
            pltpu.VMEM((C1,), jnp.int32),
            pltpu.VMEM((C1,), jnp.int32),
            pltpu.VMEM((C1,), jnp.int32),
            pltpu.VMEM((C1,), jnp.int32),
            pltpu.VMEM((C1, 128), jnp.float32),
            pltpu.VMEM((C1, 128), jnp.float32),
            pltpu.VMEM((C1, 128), jnp.float32),
            pltpu.VMEM((C1, 128), jnp.float32),
            pltpu.VMEM_SHARED((N, 128), jnp.float32),
            pltpu.SemaphoreType.DMA,
            pltpu.SemaphoreType.DMA,
            pltpu.SemaphoreType.DMA,
            pltpu.SemaphoreType.DMA,
            pltpu.SemaphoreType.DMA,
            pltpu.SemaphoreType.DMA,
        ],
    )(body)


def _make_sc2():
    def body(srcf, dstf, table_hbm, as2_hbm, ad2_hbm,
             out_hbm, den_hbm,
             sidx0, sidx1, didx0, didx1, rows0, rows1, asv, adv, den,
             exbuf, acc, isem0, isem1, rsem0, rsem1):
        cid = lax.axis_index("c")
        sid = lax.axis_index("s")
        wid = cid * NSUB + sid
        lanes = lax.iota(jnp.int32, 16)
        sidx = (sidx0, sidx1)
        didx = (didx0, didx1)
        rows = (rows0, rows1)
        isem = (isem0, isem1)
        rsem = (rsem0, rsem1)

        def issue_idx(j, b):
            c = (wid + j * NTILES) * C2
            pltpu.async_copy(srcf.at[pl.ds(c, C2)], sidx[b], isem[b])
            pltpu.async_copy(dstf.at[pl.ds(c, C2)], didx[b], isem[b])

        def wait_idx(j, b):
            c = (wid + j * NTILES) * C2
            pltpu.make_async_copy(srcf.at[pl.ds(c, C2)], sidx[b], isem[b]).wait()
            pltpu.make_async_copy(dstf.at[pl.ds(c, C2)], didx[b], isem[b]).wait()

        def issue_rows(b):
            pltpu.async_copy(table_hbm.at[sidx[b]], rows[b], rsem[b])

        def wait_rows(b):
            pltpu.make_async_copy(table_hbm.at[sidx[b]], rows[b], rsem[b]).wait()

        def valid(j):
            return (wid + j * NTILES) < NCHUNKS2

        _zero_rows(den, ZROWS)
        pltpu.sync_copy(as2_hbm, asv)
        pltpu.sync_copy(ad2_hbm, adv)
        _zero_acc(sid, den, acc)
        @pl.when(valid(0))
        def _():
            issue_idx(0, 0)
            wait_idx(0, 0)
            issue_rows(0)
        @pl.when(valid(1))
        def _():
            issue_idx(1, 1)
        plsc.subcore_barrier()

        def step(j, b):
            @pl.when(valid(j))
            def _():
                wait_rows(b)
            @pl.when(valid(j + 1))
            def _():
                wait_idx(j + 1, 1 - b)
                issue_rows(1 - b)
            @pl.when(valid(j))
            def _():
                rb = rows[b]

                def attn(g, c):
                    c0 = 16 * g
                    sv = plsc.load_gather(sidx[b], [c0 + lanes])
                    dv = plsc.load_gather(didx[b], [c0 + lanes])
                    av = plsc.load_gather(asv, [sv >> 7, sv & 127])
                    bv = plsc.load_gather(adv, [dv >> 7, dv & 127])
                    ex = _leaky_exp(av + bv)
                    plsc.addupdate_scatter(den, [dv >> 7, dv & 127], ex)
                    exbuf[pl.ds(c0, 16)] = ex
                    return c
                lax.fori_loop(0, C2 // 16, attn, 0)

                def msgs(m, c2):
                    for q in range(2):
                        c = 2 * m + q
                        exb = plsc.load_gather(
                            exbuf, [jnp.full((16,), c, jnp.int32)])
                        for t in range(F2 // 16):
                            rb[c, pl.ds(16 * t, 16)] = (
                                rb[c, pl.ds(16 * t, 16)] * exb)
                    return c2
                lax.fori_loop(0, C2 // 2, msgs, 0)

                pltpu.sync_copy(rb, acc.at[didx[b]], add=True)
            @pl.when(valid(j + 2))
            def _():
                issue_idx(j + 2, b)

        def kbody(k, carry):
            step(2 * k, 0)
            step(2 * k + 1, 1)
            return carry
        lax.fori_loop(0, (CPT2 + 1) // 2, kbody, 0)

        plsc.subcore_barrier()
        _drain_acc(cid, sid, acc, out_hbm)
        pltpu.sync_copy(den, den_hbm.at[wid])

    mesh = plsc.VectorSubcoreMesh(core_axis_name="c", subcore_axis_name="s")
    return functools.partial(
        pl.kernel,
        out_type=[jax.ShapeDtypeStruct((2, N, 128), jnp.float32),
                  jax.ShapeDtypeStruct((NTILES, ZROWS, 128), jnp.float32)],
        mesh=mesh,
        compiler_params=pltpu.CompilerParams(
            needs_layout_passes=False, use_tc_tiling_on_sc=False),
        scratch_types=[
            pltpu.VMEM((C2,), jnp.int32),
            pltpu.VMEM((C2,), jnp.int32),
            pltpu.VMEM((C2,), jnp.int32),
            pltpu.VMEM((C2,), jnp.int32),
            pltpu.VMEM((C2, 128), jnp.float32),
            pltpu.VMEM((C2, 128), jnp.float32),
            pltpu.VMEM((ZROWS, 128), jnp.float32),
            pltpu.VMEM((ZROWS, 128), jnp.float32),
            pltpu.VMEM((ZROWS, 128), jnp.float32),
            pltpu.VMEM((C2,), jnp.float32),
            pltpu.VMEM_SHARED((N, 128), jnp.float32),
            pltpu.SemaphoreType.DMA,
            pltpu.SemaphoreType.DMA,
            pltpu.SemaphoreType.DMA,
            pltpu.SemaphoreType.DMA,
        ],
    )(body)


@functools.lru_cache(maxsize=None)
def _sc_kernels():
    return (_make_sc1(), _make_sc2())


def kernel(x, seq, edges, W1, att_src1, att_dst1, bias1,
           W2, att_src2, att_dst2, bias2):
    _sc_layer1, _sc_layer2 = _sc_kernels()
    xf = x.reshape(N, ND)
    edges = edges.astype(jnp.int32)
    srcf = edges[0].reshape(E)
    dstf = edges[1].reshape(E)

    k1 = jnp.arange(F1)
    j1 = jnp.arange(H1)
    As1 = jnp.where((k1[:, None] // O1) == j1[None, :],
                    att_src1.reshape(F1)[:, None], 0.0)
    Ad1 = jnp.where((k1[:, None] // O1) == j1[None, :],
                    att_dst1.reshape(F1)[:, None], 0.0)
    a2s = att_src2.reshape(F2, 1)
    a2d = att_dst2.reshape(F2, 1)
    Eexp = ((jnp.arange(F1)[None, :] // O1) == jnp.arange(H1)[:, None]
            ).astype(jnp.float32)
    b1 = bias1.reshape(1, F1)
    b2 = bias2.reshape(1, F2)

    (table1,) = _tc_call(
        _tc1_body,
        [_rows_spec(ND), _rows_spec(1), _full_spec((ND, F1)),
         _full_spec((F1, H1)), _full_spec((F1, H1))],
        [_rows_spec(128)],
        [jax.ShapeDtypeStruct((N, 128), jnp.float32)],
        (xf, seq, W1, As1, Ad1),
        grid=(N // NC,),
    )

    acc1 = _sc_layer1(srcf, dstf, table1)

    table2, as2, ad2 = _tc_call(
        _tc2_body,
        [_acc_spec(128), _rows_spec(128), _full_spec((F1, F2)),
         _full_spec((F2, 1)), _full_spec((F2, 1)), _full_spec((1, F1)),
         _full_spec((H1, F1))],
        [_rows_spec(128), _rows_spec(1), _rows_spec(1)],
        [jax.ShapeDtypeStruct((N, 128), jnp.float32),
         jax.ShapeDtypeStruct((N, 1), jnp.float32),
         jax.ShapeDtypeStruct((N, 1), jnp.float32)],
        (acc1, table1, W2, a2s, a2d, b1, Eexp),
        grid=(N // NC,),
    )

    as2p = jnp.pad(as2.reshape(N), (0, DPAD - N)).reshape(ZROWS, 128)
    ad2p = jnp.pad(ad2.reshape(N), (0, DPAD - N)).reshape(ZROWS, 128)
    acc2, denp = _sc_layer2(srcf, dstf, table2, as2p, ad2p)

    (denr,) = _tc_call(
        _tcden_body,
        [_full_spec((NTILES, ZROWS, 128))],
        [_full_spec((ZROWS, 128))],
        [jax.ShapeDtypeStruct((ZROWS, 128), jnp.float32)],
        (denp,),
        grid=(1,),
    )
    den_n = denr.reshape(DPAD)[:N].reshape(N, 1)

    (out,) = _tc_call(
        _tc3_body,
        [_acc_spec(128), _rows_spec(128), _rows_spec(1), _rows_spec(1),
         _rows_spec(1), _full_spec((1, F2))],
        [_rows_spec(F2)],
        [jax.ShapeDtypeStruct((N, F2), jnp.float32)],
        (acc2, table2, as2, ad2, den_n, b2),
        grid=(N // NC,),
    )
    return out.reshape(NB, NC, ND)
